# baseline (speedup 1.0000x reference)
import jax
import jax.numpy as jnp
from jax.experimental import pallas as pl
from jax.experimental.pallas import tpu as pltpu


def _ipow(x, p):
    result = None
    base = x
    while p > 0:
        if p & 1:
            result = base if result is None else result * base
        p >>= 1
        if p:
            base = base * base
    return result


_NT = (((1,), (1,)), ((), ()))


def _make_body(n_total, cores, cs, cf, k_out, bug, ls, lf,
               inv_ls, inv_lf, pow_s, h_, w_, tf, ts):
    npc = n_total // cores

    def body(slow_hbm, fast_hbm, gv_hbm, gvs_hbm, ws_hbm, wf_hbm, b_hbm,
             o_hbm, s_buf, f_buf, g_buf, gs_buf, w_sb, w_fb, b_b, o_buf,
             s_sem, f_sem, misc_sem, o_sem):
        core = pl.program_id(0)
        base = core * npc

        def s_cp(i, slot):
            return pltpu.make_async_copy(
                slow_hbm.at[base + i], s_buf.at[slot], s_sem.at[slot])

        def f_cp(i, slot):
            return pltpu.make_async_copy(
                fast_hbm.at[base + i], f_buf.at[slot], f_sem.at[slot])

        g_copy = pltpu.make_async_copy(gv_hbm, g_buf, misc_sem.at[0])
        gs_copy = pltpu.make_async_copy(gvs_hbm, gs_buf, misc_sem.at[1])
        ws_copy = pltpu.make_async_copy(ws_hbm, w_sb, misc_sem.at[2])
        wf_copy = pltpu.make_async_copy(wf_hbm, w_fb, misc_sem.at[3])
        b_copy = pltpu.make_async_copy(b_hbm, b_b, misc_sem.at[4])
        for c in (g_copy, gs_copy, ws_copy, wf_copy, b_copy):
            c.start()
        s_cp(0, 0).start()
        f_cp(0, 0).start()
        for c in (g_copy, gs_copy, ws_copy, wf_copy, b_copy):
            c.wait()

        ones_row = jnp.full((1, ls), inv_ls, jnp.float32)
        lane = jax.lax.broadcasted_iota(jnp.int32, (1, cs), 1)

        def make_lhs(n):
            gf = jnp.concatenate(
                [g_buf[h, w, pl.ds(n, 1), :]
                 for h in range(h_) for w in range(w_)], axis=1)
            gs = jnp.concatenate(
                [gs_buf[h, w, pl.ds(n, 1), :]
                 for h in range(h_) for w in range(w_)], axis=1)
            gs = _ipow(gs, pow_s) * inv_ls
            return jnp.concatenate([ones_row, gs], axis=0), gf * inv_lf

        for i in range(npc):
            slot = i % 2
            if i + 1 < npc:
                s_cp(i + 1, 1 - slot).start()
                f_cp(i + 1, 1 - slot).start()
            lhs, gf = make_lhs(base + i)
            s_cp(i, slot).wait()
            f_cp(i, slot).wait()
            res = jnp.dot(lhs, s_buf[slot],
                          preferred_element_type=jnp.float32)
            sp = jnp.where(lane == bug, res[1:2, :], res[0:1, :])
            fp = jnp.dot(gf, f_buf[slot],
                         preferred_element_type=jnp.float32)
            logits = (jax.lax.dot_general(sp, w_sb[...], _NT,
                                          preferred_element_type=jnp.float32)
                      + jax.lax.dot_general(fp, w_fb[...], _NT,
                                            preferred_element_type=jnp.float32)
                      + b_b[...])
            m = jnp.max(logits, axis=-1, keepdims=True)
            e = jnp.exp(logits - m)
            o_buf[i:i + 1, :] = e / jnp.sum(e, axis=-1, keepdims=True)

        out_copy = pltpu.make_async_copy(
            o_buf, o_hbm.at[pl.ds(base, npc), :], o_sem)
        out_copy.start()
        out_copy.wait()
    return body


def kernel(slow, fast, gaze_maps, w_slow_t, w_fast_t, bias_row):
    N, Cs, Ts, H, W = slow.shape
    _, Cf, Tf, _, _ = fast.shape
    alpha = Tf // Ts
    Ls, Lf = Ts * H * W, Tf * H * W
    K = w_slow_t.shape[1]
    bug = Cf - 1
    CORES = 2

    slow_v = slow.transpose(0, 3, 4, 2, 1).reshape(N, Ls, Cs)
    fast_v = fast.transpose(0, 3, 4, 2, 1).reshape(N, Lf, Cf)
    gaze_v = gaze_maps.transpose(2, 3, 0, 1)
    gaze_vs = gaze_maps[:, ::alpha].transpose(2, 3, 0, 1)

    return pl.pallas_call(
        _make_body(N, CORES, Cs, Cf, K, bug, Ls, Lf,
                   1.0 / Ls, 1.0 / Lf, Cs, H, W, Tf, Ts),
        out_shape=jax.ShapeDtypeStruct((N, K), jnp.float32),
        grid=(CORES,),
        in_specs=[pl.BlockSpec(memory_space=pl.ANY)] * 7,
        out_specs=pl.BlockSpec(memory_space=pl.ANY),
        scratch_shapes=[
            pltpu.VMEM((2, Ls, Cs), jnp.float32),
            pltpu.VMEM((2, Lf, Cf), jnp.float32),
            pltpu.VMEM((H, W, N, Tf), jnp.float32),
            pltpu.VMEM((H, W, N, Ts), jnp.float32),
            pltpu.VMEM((K, Cs), jnp.float32),
            pltpu.VMEM((K, Cf), jnp.float32),
            pltpu.VMEM((1, K), jnp.float32),
            pltpu.VMEM((N // CORES, K), jnp.float32),
            pltpu.SemaphoreType.DMA((2,)),
            pltpu.SemaphoreType.DMA((2,)),
            pltpu.SemaphoreType.DMA((5,)),
            pltpu.SemaphoreType.DMA,
        ],
        compiler_params=pltpu.CompilerParams(
            dimension_semantics=("parallel",)),
    )(slow_v, fast_v, gaze_v, gaze_vs, w_slow_t.T, w_fast_t.T, bias_row)


# speedup vs baseline: 1.2576x; 1.2576x over previous
import jax
import jax.numpy as jnp
from jax.experimental import pallas as pl
from jax.experimental.pallas import tpu as pltpu


def _ipow(x, p):
    result = None
    base = x
    while p > 0:
        if p & 1:
            result = base if result is None else result * base
        p >>= 1
        if p:
            base = base * base
    return result


def _make_pool_body(cs, bug, inv_ls, inv_lf, pow_s, h_, w_, rows):
    def body(slow_ref, fast_ref, gv_ref, gvs_ref, sp_ref, fp_ref):
        n = pl.program_id(0)
        r = jax.lax.rem(n, rows)
        gf = jnp.concatenate(
            [gv_ref[h, w, pl.ds(n, 1), :]
             for h in range(h_) for w in range(w_)], axis=1)
        gs = jnp.concatenate(
            [gvs_ref[h, w, pl.ds(n, 1), :]
             for h in range(h_) for w in range(w_)], axis=1)
        gs = _ipow(gs, pow_s) * inv_ls
        ones_row = jnp.full((1, gs.shape[1]), inv_ls, jnp.float32)
        lhs = jnp.concatenate([ones_row, gs], axis=0)

        res = jnp.dot(lhs, slow_ref[0],
                      preferred_element_type=jnp.float32)
        lane = jax.lax.broadcasted_iota(jnp.int32, (1, cs), 1)
        sp_ref[0, pl.ds(r, 1), :] = (
            jnp.where(lane == bug, res[1:2, :], res[0:1, :]))
        fp_ref[0, pl.ds(r, 1), :] = jnp.dot(
            gf * inv_lf, fast_ref[0], preferred_element_type=jnp.float32)
    return body


_NT = (((1,), (1,)), ((), ()))


def _head_body(xs_ref, xf_ref, ws_ref, wf_ref, b_ref, o_ref):
    logits = (jax.lax.dot_general(xs_ref[...], ws_ref[...], _NT,
                                  preferred_element_type=jnp.float32)
              + jax.lax.dot_general(xf_ref[...], wf_ref[...], _NT,
                                    preferred_element_type=jnp.float32)
              + b_ref[...])
    m = jnp.max(logits, axis=-1, keepdims=True)
    e = jnp.exp(logits - m)
    o_ref[...] = e / jnp.sum(e, axis=-1, keepdims=True)


def kernel(slow, fast, gaze_maps, w_slow_t, w_fast_t, bias_row):
    N, Cs, Ts, H, W = slow.shape
    _, Cf, Tf, _, _ = fast.shape
    alpha = Tf // Ts
    Ls, Lf = Ts * H * W, Tf * H * W
    K = w_slow_t.shape[1]
    bug = Cf - 1
    ROWS = 8 if N % 8 == 0 else 1
    G = N // ROWS

    slow_v = slow.transpose(0, 3, 4, 2, 1).reshape(N, Ls, Cs)
    fast_v = fast.transpose(0, 3, 4, 2, 1).reshape(N, Lf, Cf)
    gaze_v = gaze_maps.transpose(2, 3, 0, 1)
    gaze_vs = gaze_maps[:, ::alpha].transpose(2, 3, 0, 1)

    slow_pooled, fast_pooled = pl.pallas_call(
        _make_pool_body(Cs, bug, 1.0 / Ls, 1.0 / Lf, Cs, H, W, ROWS),
        out_shape=[
            jax.ShapeDtypeStruct((G, ROWS, Cs), jnp.float32),
            jax.ShapeDtypeStruct((G, ROWS, Cf), jnp.float32),
        ],
        grid=(N,),
        in_specs=[
            pl.BlockSpec((1, Ls, Cs), lambda n: (n, 0, 0)),
            pl.BlockSpec((1, Lf, Cf), lambda n: (n, 0, 0)),
            pl.BlockSpec((H, W, N, Tf), lambda n: (0, 0, 0, 0)),
            pl.BlockSpec((H, W, N, Ts), lambda n: (0, 0, 0, 0)),
        ],
        out_specs=[
            pl.BlockSpec((1, ROWS, Cs), lambda n: (n // ROWS, 0, 0)),
            pl.BlockSpec((1, ROWS, Cf), lambda n: (n // ROWS, 0, 0)),
        ],
        compiler_params=pltpu.CompilerParams(
            dimension_semantics=("parallel",)),
    )(slow_v, fast_v, gaze_v, gaze_vs)

    return pl.pallas_call(
        _head_body,
        out_shape=jax.ShapeDtypeStruct((N, K), jnp.float32),
        grid=(1,),
        in_specs=[
            pl.BlockSpec((N, Cs), lambda i: (0, 0)),
            pl.BlockSpec((N, Cf), lambda i: (0, 0)),
            pl.BlockSpec((K, Cs), lambda i: (0, 0)),
            pl.BlockSpec((K, Cf), lambda i: (0, 0)),
            pl.BlockSpec((1, K), lambda i: (0, 0)),
        ],
        out_specs=pl.BlockSpec((N, K), lambda i: (0, 0)),
    )(slow_pooled.reshape(N, Cs), fast_pooled.reshape(N, Cf),
      w_slow_t.T, w_fast_t.T, bias_row)


# speedup vs baseline: 1.2697x; 1.0096x over previous
import jax
import jax.numpy as jnp
from jax.experimental import pallas as pl
from jax.experimental.pallas import tpu as pltpu


def _ipow(x, p):
    result = None
    base = x
    while p > 0:
        if p & 1:
            result = base if result is None else result * base
        p >>= 1
        if p:
            base = base * base
    return result


def _make_pool_body(cs, bug, inv_ls, inv_lf, pow_s, h_, w_, rows):
    def body(slow_ref, fast_ref, gv_ref, gvs_ref, sp_ref, fp_ref,
             gf_rows, gsp_rows):
        n = pl.program_id(0)
        r = jax.lax.rem(n, rows)

        @pl.when(r == 0)
        def _():
            gf_rows[...] = jnp.concatenate(
                [gv_ref[h, w, :, :] for h in range(h_) for w in range(w_)],
                axis=1) * inv_lf
            gsp_rows[...] = _ipow(
                jnp.concatenate(
                    [gvs_ref[h, w, :, :]
                     for h in range(h_) for w in range(w_)], axis=1),
                pow_s) * inv_ls

        gf = gf_rows[pl.ds(n, 1), :]
        gs = gsp_rows[pl.ds(n, 1), :]
        ones_row = jnp.full((1, gs.shape[1]), inv_ls, jnp.float32)
        lhs = jnp.concatenate([ones_row, gs], axis=0)

        res = jnp.dot(lhs, slow_ref[0],
                      preferred_element_type=jnp.float32)
        lane = jax.lax.broadcasted_iota(jnp.int32, (1, cs), 1)
        sp_ref[0, pl.ds(r, 1), :] = (
            jnp.where(lane == bug, res[1:2, :], res[0:1, :]))
        fp_ref[0, pl.ds(r, 1), :] = jnp.dot(
            gf, fast_ref[0], preferred_element_type=jnp.float32)
    return body


_NT = (((1,), (1,)), ((), ()))


def _head_body(xs_ref, xf_ref, ws_ref, wf_ref, b_ref, o_ref):
    logits = (jax.lax.dot_general(xs_ref[...], ws_ref[...], _NT,
                                  preferred_element_type=jnp.float32)
              + jax.lax.dot_general(xf_ref[...], wf_ref[...], _NT,
                                    preferred_element_type=jnp.float32)
              + b_ref[...])
    m = jnp.max(logits, axis=-1, keepdims=True)
    e = jnp.exp(logits - m)
    o_ref[...] = e / jnp.sum(e, axis=-1, keepdims=True)


def kernel(slow, fast, gaze_maps, w_slow_t, w_fast_t, bias_row):
    N, Cs, Ts, H, W = slow.shape
    _, Cf, Tf, _, _ = fast.shape
    alpha = Tf // Ts
    Ls, Lf = Ts * H * W, Tf * H * W
    K = w_slow_t.shape[1]
    bug = Cf - 1
    ROWS = 8 if N % 8 == 0 else 1
    G = N // ROWS

    slow_v = slow.transpose(0, 3, 4, 2, 1).reshape(N, Ls, Cs)
    fast_v = fast.transpose(0, 3, 4, 2, 1).reshape(N, Lf, Cf)
    gaze_v = gaze_maps.transpose(2, 3, 0, 1)
    gaze_vs = gaze_maps[:, ::alpha].transpose(2, 3, 0, 1)

    slow_pooled, fast_pooled = pl.pallas_call(
        _make_pool_body(Cs, bug, 1.0 / Ls, 1.0 / Lf, Cs, H, W, ROWS),
        out_shape=[
            jax.ShapeDtypeStruct((G, ROWS, Cs), jnp.float32),
            jax.ShapeDtypeStruct((G, ROWS, Cf), jnp.float32),
        ],
        grid=(N,),
        in_specs=[
            pl.BlockSpec((1, Ls, Cs), lambda n: (n, 0, 0)),
            pl.BlockSpec((1, Lf, Cf), lambda n: (n, 0, 0)),
            pl.BlockSpec((H, W, N, Tf), lambda n: (0, 0, 0, 0)),
            pl.BlockSpec((H, W, N, Ts), lambda n: (0, 0, 0, 0)),
        ],
        out_specs=[
            pl.BlockSpec((1, ROWS, Cs), lambda n: (n // ROWS, 0, 0)),
            pl.BlockSpec((1, ROWS, Cf), lambda n: (n // ROWS, 0, 0)),
        ],
        scratch_shapes=[
            pltpu.VMEM((N, Lf), jnp.float32),
            pltpu.VMEM((N, Ls), jnp.float32),
        ],
        compiler_params=pltpu.CompilerParams(
            dimension_semantics=("parallel",)),
    )(slow_v, fast_v, gaze_v, gaze_vs)

    return pl.pallas_call(
        _head_body,
        out_shape=jax.ShapeDtypeStruct((N, K), jnp.float32),
        grid=(1,),
        in_specs=[
            pl.BlockSpec((N, Cs), lambda i: (0, 0)),
            pl.BlockSpec((N, Cf), lambda i: (0, 0)),
            pl.BlockSpec((K, Cs), lambda i: (0, 0)),
            pl.BlockSpec((K, Cf), lambda i: (0, 0)),
            pl.BlockSpec((1, K), lambda i: (0, 0)),
        ],
        out_specs=pl.BlockSpec((N, K), lambda i: (0, 0)),
    )(slow_pooled.reshape(N, Cs), fast_pooled.reshape(N, Cf),
      w_slow_t.T, w_fast_t.T, bias_row)
